# baseline (speedup 1.0000x reference)
import jax
import jax.numpy as jnp
from jax.experimental import pallas as pl
from jax.experimental.pallas import tpu as pltpu

_N = 20000
_G = 20
_C = 21
_POS_THR, _NEG_THR = 0.5, 0.1
_TOTAL, _MAX_POS = 128, 32
_ROWS = 160
_NPAD = _ROWS * 128
_BIG = 2 ** 30
_HI0 = 0x3F800001
_STDS = (0.1, 0.1, 0.2, 0.2)


def _kernel(gt_sm, labels_sm, rpn_pl, score_t, score_T, bb_t, rpn_t, gt_t,
            cls_ref, reg_ref,
            score_s, bb_s, rpn_s, gts_s, lab_s, g_s):
    x0 = rpn_pl[0]
    y0 = rpn_pl[1]
    x1 = rpn_pl[2]
    y1 = rpn_pl[3]
    area_a = (x1 - x0) * (y1 - y0)

    mx = jnp.full((_ROWS, 128), -1.0, dtype=jnp.float32)
    gi = jnp.zeros((_ROWS, 128), dtype=jnp.int32)
    for g in range(_G):
        bx0 = gt_sm[g, 0]
        by0 = gt_sm[g, 1]
        bx1 = gt_sm[g, 2]
        by1 = gt_sm[g, 3]
        area_b = (bx1 - bx0) * (by1 - by0)
        iw = jnp.clip(jnp.minimum(x1, bx1) - jnp.maximum(x0, bx0), 0.0, None)
        ih = jnp.clip(jnp.minimum(y1, by1) - jnp.maximum(y0, by0), 0.0, None)
        inter = iw * ih
        iou = inter / (area_a + area_b - inter + 1e-8)
        upd = iou > mx
        mx = jnp.where(upd, iou, mx)
        gi = jnp.where(upd, g, gi)

    idx = (jax.lax.broadcasted_iota(jnp.int32, (_ROWS, 128), 0) * 128
           + jax.lax.broadcasted_iota(jnp.int32, (_ROWS, 128), 1))
    mx = jnp.where(idx < _N, mx, 0.3)
    key = idx * 32 + gi

    pos_mask = mx >= _POS_THR
    num_pos = jnp.minimum(jnp.sum(pos_mask.astype(jnp.int32)), _MAX_POS)

    score_s[...] = jnp.zeros((_MAX_POS, _C), jnp.float32)
    bb_s[...] = jnp.zeros((_MAX_POS, _C * 4), jnp.float32)
    rpn_s[...] = jnp.zeros((_MAX_POS, 4), jnp.float32)
    gts_s[...] = jnp.zeros((_MAX_POS, 4), jnp.float32)
    lab_s[...] = jnp.zeros((_MAX_POS, 1), jnp.float32)
    g_s[...] = jnp.zeros((_MAX_POS, 1), jnp.float32)

    def pos_body(r, score):
        m = jnp.max(score)
        km = jnp.min(jnp.where(score == m, key, _BIG))
        pick = km // 32
        gpick = km - pick * 32
        score = jnp.where(key == km, -2.0, score)
        score_s[pl.ds(r, 1), :] = score_t[pl.ds(pick, 1), :]
        bb_s[pl.ds(r, 1), :] = bb_t[pl.ds(pick, 1), :]
        rpn_s[pl.ds(r, 1), :] = rpn_t[pl.ds(pick, 1), :]
        gts_s[pl.ds(r, 1), :] = gt_t[pl.ds(gpick, 1), :]
        lab_s[pl.ds(r, 1), :] = jnp.full(
            (1, 1), labels_sm[gpick], jnp.int32).astype(jnp.float32)
        g_s[pl.ds(r, 1), :] = jnp.full((1, 1), gpick, jnp.int32).astype(
            jnp.float32)
        return score

    pos_score = jnp.where(pos_mask, mx, -1.0)
    jax.lax.fori_loop(0, num_pos, pos_body, pos_score)

    needed = _TOTAL - num_pos
    neg_score = jnp.where(mx < _NEG_THR, 1.0 - mx, -1.0)
    zero = num_pos * 0
    t = jnp.where(neg_score < 0.0, zero - 1,
                  jax.lax.bitcast_convert_type(neg_score, jnp.int32))

    def vsearch(i, lohi):
        lo, hi = lohi
        mid = lo + (hi - lo) // 2
        cnt = jnp.sum(jnp.where(t >= mid, 1, 0))
        ok = cnt >= needed
        return (jnp.where(ok, mid, lo), jnp.where(ok, hi, mid))

    thr, _ = jax.lax.fori_loop(0, 31, vsearch, (zero - 1, zero + _HI0))
    c_gt = jnp.sum(jnp.where(t > thr, 1, 0))
    r_tie = needed - c_gt
    tie = t == thr

    def isearch(i, lohi):
        lo, hi = lohi
        mid = lo + (hi - lo) // 2
        cnt = jnp.sum(jnp.where(tie & (idx < mid), 1, 0))
        ok = cnt >= r_tie
        return (jnp.where(ok, lo, mid), jnp.where(ok, mid, hi))

    _, cut = jax.lax.fori_loop(0, 15, isearch, (zero, zero + _NPAD))
    neg_sel = (t > thr) | (tie & (idx < cut))

    planes = [score_T[c] for c in range(_C)]
    m2 = planes[0]
    for c in range(1, _C):
        m2 = jnp.maximum(m2, planes[c])
    ssum = jnp.zeros((_ROWS, 128), jnp.float32)
    for c in range(_C):
        ssum = ssum + jnp.exp(planes[c] - m2)
    lse_d = jnp.log(ssum) + m2
    neg_cls = jnp.sum(jnp.where(neg_sel, lse_d - planes[0], 0.0))

    s = score_s[...]
    m3 = jnp.max(s, axis=1, keepdims=True)
    e = jnp.exp(s - m3)
    lse = jnp.log(jnp.sum(e, axis=1, keepdims=True)) + m3
    lab = lab_s[...].astype(jnp.int32)
    cl = jax.lax.broadcasted_iota(jnp.int32, (_MAX_POS, _C), 1)
    picked = jnp.sum(jnp.where(cl == lab, s, 0.0), axis=1, keepdims=True)
    slot = jax.lax.broadcasted_iota(jnp.int32, (_MAX_POS, 1), 0)
    pvalid = slot < num_pos
    cls_sum = jnp.sum(jnp.where(pvalid, lse - picked, 0.0)) + neg_cls

    bb = bb_s[...]
    gv = g_s[...].astype(jnp.int32)
    lane = jax.lax.broadcasted_iota(jnp.int32, (_MAX_POS, _C * 4), 1)
    pred = jnp.concatenate(
        [jnp.sum(jnp.where(lane == gv * 4 + c, bb, 0.0), axis=1,
                 keepdims=True) for c in range(4)], axis=1)
    p = rpn_s[...]
    q = gts_s[...]
    pw = p[:, 2:3] - p[:, 0:1]
    ph = p[:, 3:4] - p[:, 1:2]
    pcx = p[:, 0:1] + 0.5 * pw
    pcy = p[:, 1:2] + 0.5 * ph
    gw = q[:, 2:3] - q[:, 0:1]
    gh = q[:, 3:4] - q[:, 1:2]
    gcx = q[:, 0:1] + 0.5 * gw
    gcy = q[:, 1:2] + 0.5 * gh
    tx = (gcx - pcx) / (pw + 1e-8) / _STDS[0]
    ty = (gcy - pcy) / (ph + 1e-8) / _STDS[1]
    tw = jnp.log(jnp.clip(gw, 1e-6, None) / jnp.clip(pw, 1e-6, None)) / _STDS[2]
    th = jnp.log(jnp.clip(gh, 1e-6, None) / jnp.clip(ph, 1e-6, None)) / _STDS[3]
    tt = jnp.concatenate([tx, ty, tw, th], axis=1)
    diff = pred - tt
    ad = jnp.abs(diff)
    sl1 = jnp.where(ad < 1.0, 0.5 * diff * diff, ad - 0.5)
    reg_sum = jnp.sum(jnp.where(pvalid, jnp.sum(sl1, axis=1, keepdims=True),
                                0.0))

    cls_ref[...] = jnp.full((1, 128), cls_sum / _TOTAL, jnp.float32)
    reg_ref[...] = jnp.full((1, 128), reg_sum / _TOTAL, jnp.float32)


@jax.jit
def kernel(image_shape, rpn_proposals_bboxes, roi_score, roi_bboxes_txtytwth,
           gt_bboxes, gt_labels):
    del image_shape
    rpn_pad = jnp.pad(rpn_proposals_bboxes, ((0, _NPAD - _N), (0, 0)))
    rpn_pl = rpn_pad.T.reshape(4, _ROWS, 128)
    score_T = jnp.pad(roi_score, ((0, _NPAD - _N), (0, 0))).T.reshape(
        _C, _ROWS, 128)

    cls_out, reg_out = pl.pallas_call(
        _kernel,
        in_specs=[
            pl.BlockSpec(memory_space=pltpu.SMEM),
            pl.BlockSpec(memory_space=pltpu.SMEM),
        ] + [pl.BlockSpec(memory_space=pltpu.VMEM)] * 6,
        out_specs=[pl.BlockSpec(memory_space=pltpu.VMEM)] * 2,
        out_shape=[jax.ShapeDtypeStruct((1, 128), jnp.float32)] * 2,
        scratch_shapes=[
            pltpu.VMEM((_MAX_POS, _C), jnp.float32),
            pltpu.VMEM((_MAX_POS, _C * 4), jnp.float32),
            pltpu.VMEM((_MAX_POS, 4), jnp.float32),
            pltpu.VMEM((_MAX_POS, 4), jnp.float32),
            pltpu.VMEM((_MAX_POS, 1), jnp.float32),
            pltpu.VMEM((_MAX_POS, 1), jnp.float32),
        ],
    )(gt_bboxes, gt_labels, rpn_pl, roi_score, score_T,
      roi_bboxes_txtytwth.reshape(_N, _C * 4), rpn_proposals_bboxes,
      gt_bboxes)

    return (cls_out[0, 0], reg_out[0, 0])


# speedup vs baseline: 1.0346x; 1.0346x over previous
import jax
import jax.numpy as jnp
from jax.experimental import pallas as pl
from jax.experimental.pallas import tpu as pltpu

_N = 20000
_G = 20
_C = 21
_POS_THR, _NEG_THR = 0.5, 0.1
_TOTAL, _MAX_POS = 128, 32
_ROWS = 160
_NPAD = _ROWS * 128
_BIG = 2 ** 30
_HI0 = 0x3F800001
_STDS = (0.1, 0.1, 0.2, 0.2)


def _kernel(gt_sm, labels_sm, rpn_pl, score_t, score_T, bb_t, rpn_t, gt_t,
            cls_ref, reg_ref,
            score_s, bb_s, rpn_s, gts_s, lab_s, g_s, dma_sem):
    x0 = rpn_pl[0]
    y0 = rpn_pl[1]
    x1 = rpn_pl[2]
    y1 = rpn_pl[3]
    area_a = (x1 - x0) * (y1 - y0)

    mx = jnp.full((_ROWS, 128), -1.0, dtype=jnp.float32)
    gi = jnp.zeros((_ROWS, 128), dtype=jnp.int32)
    for g in range(_G):
        bx0 = gt_sm[g, 0]
        by0 = gt_sm[g, 1]
        bx1 = gt_sm[g, 2]
        by1 = gt_sm[g, 3]
        area_b = (bx1 - bx0) * (by1 - by0)
        iw = jnp.clip(jnp.minimum(x1, bx1) - jnp.maximum(x0, bx0), 0.0, None)
        ih = jnp.clip(jnp.minimum(y1, by1) - jnp.maximum(y0, by0), 0.0, None)
        inter = iw * ih
        iou = inter / (area_a + area_b - inter + 1e-8)
        upd = iou > mx
        mx = jnp.where(upd, iou, mx)
        gi = jnp.where(upd, g, gi)

    idx = (jax.lax.broadcasted_iota(jnp.int32, (_ROWS, 128), 0) * 128
           + jax.lax.broadcasted_iota(jnp.int32, (_ROWS, 128), 1))
    mx = jnp.where(idx < _N, mx, 0.3)
    key = idx * 32 + gi

    pos_mask = mx >= _POS_THR
    num_pos = jnp.minimum(jnp.sum(pos_mask.astype(jnp.int32)), _MAX_POS)

    score_s[...] = jnp.zeros((_MAX_POS, _C), jnp.float32)
    bb_s[...] = jnp.zeros((_MAX_POS, _C * 4), jnp.float32)
    rpn_s[...] = jnp.zeros((_MAX_POS, 4), jnp.float32)
    gts_s[...] = jnp.zeros((_MAX_POS, 4), jnp.float32)
    lab_s[...] = jnp.zeros((_MAX_POS, 1), jnp.float32)
    g_s[...] = jnp.zeros((_MAX_POS, 1), jnp.float32)

    def pos_body(r, score):
        m = jnp.max(score)
        km = jnp.min(jnp.where(score == m, key, _BIG))
        pick = km // 32
        gpick = km - pick * 32
        score = jnp.where(key == km, -2.0, score)
        score_s[pl.ds(r, 1), :] = score_t[pl.ds(pick, 1), :]
        pltpu.make_async_copy(bb_t.at[pl.ds(pick, 1), :],
                              bb_s.at[pl.ds(r, 1), :], dma_sem).start()
        rpn_s[pl.ds(r, 1), :] = rpn_t[pl.ds(pick, 1), :]
        gts_s[pl.ds(r, 1), :] = gt_t[pl.ds(gpick, 1), :]
        lab_s[pl.ds(r, 1), :] = jnp.full(
            (1, 1), labels_sm[gpick], jnp.int32).astype(jnp.float32)
        g_s[pl.ds(r, 1), :] = jnp.full((1, 1), gpick, jnp.int32).astype(
            jnp.float32)
        return score

    pos_score = jnp.where(pos_mask, mx, -1.0)
    jax.lax.fori_loop(0, num_pos, pos_body, pos_score)

    def drain_body(r, c):
        pltpu.make_async_copy(bb_t.at[pl.ds(0, 1), :],
                              bb_s.at[pl.ds(0, 1), :], dma_sem).wait()
        return c

    jax.lax.fori_loop(0, num_pos, drain_body, 0)

    needed = _TOTAL - num_pos
    neg_score = jnp.where(mx < _NEG_THR, 1.0 - mx, -1.0)
    zero = num_pos * 0
    t = jnp.where(neg_score < 0.0, zero - 1,
                  jax.lax.bitcast_convert_type(neg_score, jnp.int32))

    def vsearch(i, lohi):
        lo, hi = lohi
        mid = lo + (hi - lo) // 2
        cnt = jnp.sum(jnp.where(t >= mid, 1, 0))
        ok = cnt >= needed
        return (jnp.where(ok, mid, lo), jnp.where(ok, hi, mid))

    thr, _ = jax.lax.fori_loop(0, 31, vsearch, (zero - 1, zero + _HI0))
    c_gt = jnp.sum(jnp.where(t > thr, 1, 0))
    r_tie = needed - c_gt
    tie = t == thr

    def isearch(i, lohi):
        lo, hi = lohi
        mid = lo + (hi - lo) // 2
        cnt = jnp.sum(jnp.where(tie & (idx < mid), 1, 0))
        ok = cnt >= r_tie
        return (jnp.where(ok, lo, mid), jnp.where(ok, mid, hi))

    _, cut = jax.lax.fori_loop(0, 15, isearch, (zero, zero + _NPAD))
    neg_sel = (t > thr) | (tie & (idx < cut))

    planes = [score_T[c] for c in range(_C)]
    m2 = planes[0]
    for c in range(1, _C):
        m2 = jnp.maximum(m2, planes[c])
    ssum = jnp.zeros((_ROWS, 128), jnp.float32)
    for c in range(_C):
        ssum = ssum + jnp.exp(planes[c] - m2)
    lse_d = jnp.log(ssum) + m2
    neg_cls = jnp.sum(jnp.where(neg_sel, lse_d - planes[0], 0.0))

    s = score_s[...]
    m3 = jnp.max(s, axis=1, keepdims=True)
    e = jnp.exp(s - m3)
    lse = jnp.log(jnp.sum(e, axis=1, keepdims=True)) + m3
    lab = lab_s[...].astype(jnp.int32)
    cl = jax.lax.broadcasted_iota(jnp.int32, (_MAX_POS, _C), 1)
    picked = jnp.sum(jnp.where(cl == lab, s, 0.0), axis=1, keepdims=True)
    slot = jax.lax.broadcasted_iota(jnp.int32, (_MAX_POS, 1), 0)
    pvalid = slot < num_pos
    cls_sum = jnp.sum(jnp.where(pvalid, lse - picked, 0.0)) + neg_cls

    bb = bb_s[...]
    gv = g_s[...].astype(jnp.int32)
    lane = jax.lax.broadcasted_iota(jnp.int32, (_MAX_POS, _C * 4), 1)
    pred = jnp.concatenate(
        [jnp.sum(jnp.where(lane == gv * 4 + c, bb, 0.0), axis=1,
                 keepdims=True) for c in range(4)], axis=1)
    p = rpn_s[...]
    q = gts_s[...]
    pw = p[:, 2:3] - p[:, 0:1]
    ph = p[:, 3:4] - p[:, 1:2]
    pcx = p[:, 0:1] + 0.5 * pw
    pcy = p[:, 1:2] + 0.5 * ph
    gw = q[:, 2:3] - q[:, 0:1]
    gh = q[:, 3:4] - q[:, 1:2]
    gcx = q[:, 0:1] + 0.5 * gw
    gcy = q[:, 1:2] + 0.5 * gh
    tx = (gcx - pcx) / (pw + 1e-8) / _STDS[0]
    ty = (gcy - pcy) / (ph + 1e-8) / _STDS[1]
    tw = jnp.log(jnp.clip(gw, 1e-6, None) / jnp.clip(pw, 1e-6, None)) / _STDS[2]
    th = jnp.log(jnp.clip(gh, 1e-6, None) / jnp.clip(ph, 1e-6, None)) / _STDS[3]
    tt = jnp.concatenate([tx, ty, tw, th], axis=1)
    diff = pred - tt
    ad = jnp.abs(diff)
    sl1 = jnp.where(ad < 1.0, 0.5 * diff * diff, ad - 0.5)
    reg_sum = jnp.sum(jnp.where(pvalid, jnp.sum(sl1, axis=1, keepdims=True),
                                0.0))

    cls_ref[...] = jnp.full((1, 128), cls_sum / _TOTAL, jnp.float32)
    reg_ref[...] = jnp.full((1, 128), reg_sum / _TOTAL, jnp.float32)


@jax.jit
def kernel(image_shape, rpn_proposals_bboxes, roi_score, roi_bboxes_txtytwth,
           gt_bboxes, gt_labels):
    del image_shape
    rpn_pad = jnp.pad(rpn_proposals_bboxes, ((0, _NPAD - _N), (0, 0)))
    rpn_pl = rpn_pad.T.reshape(4, _ROWS, 128)
    score_T = jnp.pad(roi_score, ((0, _NPAD - _N), (0, 0))).T.reshape(
        _C, _ROWS, 128)

    cls_out, reg_out = pl.pallas_call(
        _kernel,
        in_specs=[
            pl.BlockSpec(memory_space=pltpu.SMEM),
            pl.BlockSpec(memory_space=pltpu.SMEM),
            pl.BlockSpec(memory_space=pltpu.VMEM),
            pl.BlockSpec(memory_space=pltpu.VMEM),
            pl.BlockSpec(memory_space=pltpu.VMEM),
            pl.BlockSpec(memory_space=pltpu.HBM),
            pl.BlockSpec(memory_space=pltpu.VMEM),
            pl.BlockSpec(memory_space=pltpu.VMEM),
        ],
        out_specs=[pl.BlockSpec(memory_space=pltpu.VMEM)] * 2,
        out_shape=[jax.ShapeDtypeStruct((1, 128), jnp.float32)] * 2,
        scratch_shapes=[
            pltpu.VMEM((_MAX_POS, _C), jnp.float32),
            pltpu.VMEM((_MAX_POS, _C * 4), jnp.float32),
            pltpu.VMEM((_MAX_POS, 4), jnp.float32),
            pltpu.VMEM((_MAX_POS, 4), jnp.float32),
            pltpu.VMEM((_MAX_POS, 1), jnp.float32),
            pltpu.VMEM((_MAX_POS, 1), jnp.float32),
            pltpu.SemaphoreType.DMA,
        ],
    )(gt_bboxes, gt_labels, rpn_pl, roi_score, score_T,
      roi_bboxes_txtytwth.reshape(_N, _C * 4), rpn_proposals_bboxes,
      gt_bboxes)

    return (cls_out[0, 0], reg_out[0, 0])


# speedup vs baseline: 1.0833x; 1.0471x over previous
import jax
import jax.numpy as jnp
from jax.experimental import pallas as pl
from jax.experimental.pallas import tpu as pltpu

_N = 20000
_G = 20
_C = 21
_POS_THR, _NEG_THR = 0.5, 0.1
_TOTAL, _MAX_POS = 128, 32
_ROWS = 160
_NPAD = _ROWS * 128
_BIG = 2 ** 30
_HI0 = 0x3F800001
_STDS = (0.1, 0.1, 0.2, 0.2)


def _kernel(gt_sm, labels_sm, rpn_pl, score_t, score_T, bb_t, rpn_t, gt_t,
            cls_ref, reg_ref,
            score_s, bb_s, rpn_s, gts_s, lab_s, g_s, dma_sem, sc_sem):
    x0 = rpn_pl[0]
    y0 = rpn_pl[1]
    x1 = rpn_pl[2]
    y1 = rpn_pl[3]
    area_a = (x1 - x0) * (y1 - y0)

    mx = jnp.full((_ROWS, 128), -1.0, dtype=jnp.float32)
    gi = jnp.zeros((_ROWS, 128), dtype=jnp.int32)
    for g in range(_G):
        bx0 = gt_sm[g, 0]
        by0 = gt_sm[g, 1]
        bx1 = gt_sm[g, 2]
        by1 = gt_sm[g, 3]
        area_b = (bx1 - bx0) * (by1 - by0)
        iw = jnp.clip(jnp.minimum(x1, bx1) - jnp.maximum(x0, bx0), 0.0, None)
        ih = jnp.clip(jnp.minimum(y1, by1) - jnp.maximum(y0, by0), 0.0, None)
        inter = iw * ih
        iou = inter / (area_a + area_b - inter + 1e-8)
        upd = iou > mx
        mx = jnp.where(upd, iou, mx)
        gi = jnp.where(upd, g, gi)

    idx = (jax.lax.broadcasted_iota(jnp.int32, (_ROWS, 128), 0) * 128
           + jax.lax.broadcasted_iota(jnp.int32, (_ROWS, 128), 1))
    mx = jnp.where(idx < _N, mx, 0.3)
    key = idx * 32 + gi

    pos_mask = mx >= _POS_THR
    num_pos = jnp.minimum(jnp.sum(pos_mask.astype(jnp.int32)), _MAX_POS)

    score_s[...] = jnp.zeros((_MAX_POS, _C), jnp.float32)
    bb_s[...] = jnp.zeros((_MAX_POS, _C * 4), jnp.float32)
    rpn_s[...] = jnp.zeros((_MAX_POS, 4), jnp.float32)
    gts_s[...] = jnp.zeros((_MAX_POS, 4), jnp.float32)
    lab_s[...] = jnp.zeros((_MAX_POS, 1), jnp.float32)
    g_s[...] = jnp.zeros((_MAX_POS, 1), jnp.float32)

    def pos_body(r, score):
        m = jnp.max(score)
        km = jnp.min(jnp.where(score == m, key, _BIG))
        pick = km // 32
        gpick = km - pick * 32
        score = jnp.where(key == km, -2.0, score)
        pltpu.make_async_copy(score_t.at[pl.ds(pick, 1), :],
                              score_s.at[pl.ds(r, 1), :], sc_sem).start()
        pltpu.make_async_copy(bb_t.at[pl.ds(pick, 1), :],
                              bb_s.at[pl.ds(r, 1), :], dma_sem).start()
        rpn_s[pl.ds(r, 1), :] = rpn_t[pl.ds(pick, 1), :]
        gts_s[pl.ds(r, 1), :] = gt_t[pl.ds(gpick, 1), :]
        lab_s[pl.ds(r, 1), :] = jnp.full(
            (1, 1), labels_sm[gpick], jnp.int32).astype(jnp.float32)
        g_s[pl.ds(r, 1), :] = jnp.full((1, 1), gpick, jnp.int32).astype(
            jnp.float32)
        return score

    pos_score = jnp.where(pos_mask, mx, -1.0)
    jax.lax.fori_loop(0, num_pos, pos_body, pos_score)

    def drain_body(r, c):
        pltpu.make_async_copy(bb_t.at[pl.ds(0, 1), :],
                              bb_s.at[pl.ds(0, 1), :], dma_sem).wait()
        pltpu.make_async_copy(score_t.at[pl.ds(0, 1), :],
                              score_s.at[pl.ds(0, 1), :], sc_sem).wait()
        return c

    jax.lax.fori_loop(0, num_pos, drain_body, 0)

    needed = _TOTAL - num_pos
    neg_score = jnp.where(mx < _NEG_THR, 1.0 - mx, -1.0)
    zero = num_pos * 0
    t = jnp.where(neg_score < 0.0, zero - 1,
                  jax.lax.bitcast_convert_type(neg_score, jnp.int32))

    def vsearch(i, lohi):
        lo, hi = lohi
        mid = lo + (hi - lo) // 2
        cnt = jnp.sum(jnp.where(t >= mid, 1, 0))
        ok = cnt >= needed
        return (jnp.where(ok, mid, lo), jnp.where(ok, hi, mid))

    thr, _ = jax.lax.fori_loop(0, 31, vsearch, (zero - 1, zero + _HI0))
    c_gt = jnp.sum(jnp.where(t > thr, 1, 0))
    r_tie = needed - c_gt
    tie = t == thr

    def isearch(i, lohi):
        lo, hi = lohi
        mid = lo + (hi - lo) // 2
        cnt = jnp.sum(jnp.where(tie & (idx < mid), 1, 0))
        ok = cnt >= r_tie
        return (jnp.where(ok, lo, mid), jnp.where(ok, mid, hi))

    _, cut = jax.lax.fori_loop(0, 15, isearch, (zero, zero + _NPAD))
    neg_sel = (t > thr) | (tie & (idx < cut))

    planes = [score_T[c] for c in range(_C)]
    m2 = planes[0]
    for c in range(1, _C):
        m2 = jnp.maximum(m2, planes[c])
    ssum = jnp.zeros((_ROWS, 128), jnp.float32)
    for c in range(_C):
        ssum = ssum + jnp.exp(planes[c] - m2)
    lse_d = jnp.log(ssum) + m2
    neg_cls = jnp.sum(jnp.where(neg_sel, lse_d - planes[0], 0.0))

    s = score_s[...]
    m3 = jnp.max(s, axis=1, keepdims=True)
    e = jnp.exp(s - m3)
    lse = jnp.log(jnp.sum(e, axis=1, keepdims=True)) + m3
    lab = lab_s[...].astype(jnp.int32)
    cl = jax.lax.broadcasted_iota(jnp.int32, (_MAX_POS, _C), 1)
    picked = jnp.sum(jnp.where(cl == lab, s, 0.0), axis=1, keepdims=True)
    slot = jax.lax.broadcasted_iota(jnp.int32, (_MAX_POS, 1), 0)
    pvalid = slot < num_pos
    cls_sum = jnp.sum(jnp.where(pvalid, lse - picked, 0.0)) + neg_cls

    bb = bb_s[...]
    gv = g_s[...].astype(jnp.int32)
    lane = jax.lax.broadcasted_iota(jnp.int32, (_MAX_POS, _C * 4), 1)
    pred = jnp.concatenate(
        [jnp.sum(jnp.where(lane == gv * 4 + c, bb, 0.0), axis=1,
                 keepdims=True) for c in range(4)], axis=1)
    p = rpn_s[...]
    q = gts_s[...]
    pw = p[:, 2:3] - p[:, 0:1]
    ph = p[:, 3:4] - p[:, 1:2]
    pcx = p[:, 0:1] + 0.5 * pw
    pcy = p[:, 1:2] + 0.5 * ph
    gw = q[:, 2:3] - q[:, 0:1]
    gh = q[:, 3:4] - q[:, 1:2]
    gcx = q[:, 0:1] + 0.5 * gw
    gcy = q[:, 1:2] + 0.5 * gh
    tx = (gcx - pcx) / (pw + 1e-8) / _STDS[0]
    ty = (gcy - pcy) / (ph + 1e-8) / _STDS[1]
    tw = jnp.log(jnp.clip(gw, 1e-6, None) / jnp.clip(pw, 1e-6, None)) / _STDS[2]
    th = jnp.log(jnp.clip(gh, 1e-6, None) / jnp.clip(ph, 1e-6, None)) / _STDS[3]
    tt = jnp.concatenate([tx, ty, tw, th], axis=1)
    diff = pred - tt
    ad = jnp.abs(diff)
    sl1 = jnp.where(ad < 1.0, 0.5 * diff * diff, ad - 0.5)
    reg_sum = jnp.sum(jnp.where(pvalid, jnp.sum(sl1, axis=1, keepdims=True),
                                0.0))

    cls_ref[...] = jnp.full((1, 128), cls_sum / _TOTAL, jnp.float32)
    reg_ref[...] = jnp.full((1, 128), reg_sum / _TOTAL, jnp.float32)


@jax.jit
def kernel(image_shape, rpn_proposals_bboxes, roi_score, roi_bboxes_txtytwth,
           gt_bboxes, gt_labels):
    del image_shape
    rpn_pad = jnp.pad(rpn_proposals_bboxes, ((0, _NPAD - _N), (0, 0)))
    rpn_pl = rpn_pad.T.reshape(4, _ROWS, 128)
    score_T = jnp.pad(roi_score, ((0, _NPAD - _N), (0, 0))).T.reshape(
        _C, _ROWS, 128)

    cls_out, reg_out = pl.pallas_call(
        _kernel,
        in_specs=[
            pl.BlockSpec(memory_space=pltpu.SMEM),
            pl.BlockSpec(memory_space=pltpu.SMEM),
            pl.BlockSpec(memory_space=pltpu.VMEM),
            pl.BlockSpec(memory_space=pltpu.HBM),
            pl.BlockSpec(memory_space=pltpu.VMEM),
            pl.BlockSpec(memory_space=pltpu.HBM),
            pl.BlockSpec(memory_space=pltpu.VMEM),
            pl.BlockSpec(memory_space=pltpu.VMEM),
        ],
        out_specs=[pl.BlockSpec(memory_space=pltpu.VMEM)] * 2,
        out_shape=[jax.ShapeDtypeStruct((1, 128), jnp.float32)] * 2,
        scratch_shapes=[
            pltpu.VMEM((_MAX_POS, _C), jnp.float32),
            pltpu.VMEM((_MAX_POS, _C * 4), jnp.float32),
            pltpu.VMEM((_MAX_POS, 4), jnp.float32),
            pltpu.VMEM((_MAX_POS, 4), jnp.float32),
            pltpu.VMEM((_MAX_POS, 1), jnp.float32),
            pltpu.VMEM((_MAX_POS, 1), jnp.float32),
            pltpu.SemaphoreType.DMA,
            pltpu.SemaphoreType.DMA,
        ],
    )(gt_bboxes, gt_labels, rpn_pl, roi_score, score_T,
      roi_bboxes_txtytwth.reshape(_N, _C * 4), rpn_proposals_bboxes,
      gt_bboxes)

    return (cls_out[0, 0], reg_out[0, 0])
